# baseline (speedup 1.0000x reference)
import jax
import jax.numpy as jnp
import kernel_tc
import kernel_sc

_SPLIT = 57344


def kernel(inputs, inputs_positions, position_emb):
    B, N, D = inputs.shape
    tot = B * N
    x = inputs.reshape(tot, D)
    pos = inputs_positions.reshape(tot)

    s_b = _SPLIT // N
    out_tc = kernel_tc.kernel(
        x[:_SPLIT].reshape(s_b, N, D), pos[:_SPLIT].reshape(s_b, N),
        position_emb)
    out_sc = kernel_sc.kernel(
        x[_SPLIT:].reshape(B - s_b, N, D), pos[_SPLIT:].reshape(B - s_b, N),
        position_emb)
    return jnp.concatenate([out_tc, out_sc], axis=0)


# speedup vs baseline: 3.3827x; 3.3827x over previous
import jax
import jax.numpy as jnp
from jax.experimental import pallas as pl

_BLOCK = 8192
_TPAD = 128


def _body(pos_ref, x_ref, tab_ref, o_ref):
    idx = pos_ref[0, 0, :]
    cols = jax.lax.broadcasted_iota(jnp.int32, (1, _TPAD), 1)
    onehot = (idx[:, None] == cols).astype(jnp.float32)
    g = jax.lax.dot_general(
        onehot, tab_ref[...], (((1,), (0,)), ((), ())),
        preferred_element_type=jnp.float32)
    o_ref[...] = x_ref[...] + g


def kernel(inputs, inputs_positions, position_emb):
    B, N, D = inputs.shape
    tot = B * N
    nb = tot // _BLOCK
    x = inputs.reshape(tot, D)
    pos = inputs_positions.reshape(nb, 1, _BLOCK).astype(jnp.int32)
    table = jnp.squeeze(position_emb, axis=0)
    table = jnp.pad(table, ((0, _TPAD - table.shape[0]), (0, 0)))

    out = pl.pallas_call(
        _body,
        grid=(nb,),
        in_specs=[
            pl.BlockSpec((1, 1, _BLOCK), lambda i: (i, 0, 0)),
            pl.BlockSpec((_BLOCK, D), lambda i: (i, 0)),
            pl.BlockSpec((_TPAD, D), lambda i: (0, 0)),
        ],
        out_specs=pl.BlockSpec((_BLOCK, D), lambda i: (i, 0)),
        out_shape=jax.ShapeDtypeStruct((tot, D), jnp.float32),
    )(pos, x, table)
    return out.reshape(B, N, D)
